# Initial kernel scaffold; baseline (speedup 1.0000x reference)
import functools

import jax
import jax.numpy as jnp
from jax import lax
from jax.experimental import pallas as pl
from jax.experimental.pallas import tpu as pltpu
from jax.experimental.pallas import tpu_sc as plsc

B = 16384
V = 100000
H = 128
D = 64


_NC = 2
_NS = 16
_NW = _NC * _NS
_BPW = B // _NW
_CHUNK = 128
_NCH = _BPW // _CHUNK


def _gather_body(x0_hbm, x1_hbm, e0_hbm, e1_hbm, out0, out1,
                 idx0_v, idx1_v, r0, r1, r2, r3, sem):
    wid = lax.axis_index("s") * _NC + lax.axis_index("c")
    rows = (r0, r1, r2, r3)
    pltpu.sync_copy(x0_hbm.at[pl.ds(wid * _NCH, _NCH)], idx0_v)
    pltpu.sync_copy(x1_hbm.at[pl.ds(wid * _NCH, _NCH)], idx1_v)
    base = wid * _BPW
    for table, idx_v, out in ((e0_hbm, idx0_v, out0), (e1_hbm, idx1_v, out1)):
        copies = [
            pltpu.async_copy(table.at[idx_v.at[j]], rows[j], sem)
            for j in range(_NCH)
        ]
        for c in copies:
            c.wait()
        for j in range(_NCH):
            pltpu.sync_copy(rows[j], out.at[pl.ds(base + j * _CHUNK, _CHUNK)])


@functools.partial(
    pl.kernel,
    mesh=plsc.VectorSubcoreMesh(core_axis_name="c", subcore_axis_name="s"),
    out_type=[
        jax.ShapeDtypeStruct((B, H), jnp.float32),
        jax.ShapeDtypeStruct((B, H), jnp.float32),
    ],
    scratch_types=[
        pltpu.VMEM((_NCH, _CHUNK), jnp.int32),
        pltpu.VMEM((_NCH, _CHUNK), jnp.int32),
        pltpu.VMEM((_CHUNK, H), jnp.float32),
        pltpu.VMEM((_CHUNK, H), jnp.float32),
        pltpu.VMEM((_CHUNK, H), jnp.float32),
        pltpu.VMEM((_CHUNK, H), jnp.float32),
        pltpu.SemaphoreType.DMA,
    ],
)
def _sc_gather(x0_hbm, x1_hbm, e0_hbm, e1_hbm, out0, out1,
               idx0_v, idx1_v, r0, r1, r2, r3, sem):
    _gather_body(x0_hbm, x1_hbm, e0_hbm, e1_hbm, out0, out1,
                 idx0_v, idx1_v, r0, r1, r2, r3, sem)


_BS = 2048


def _silu(v):
    return v * jax.nn.sigmoid(v)


def _mlp_body(e0_ref, e1_ref, w1a_ref, w1b_ref, w2_ref, w3_ref,
              b1_ref, b2_ref, b3_ref, out_ref):
    z = jnp.dot(e0_ref[...], w1a_ref[...], preferred_element_type=jnp.float32)
    z += jnp.dot(e1_ref[...], w1b_ref[...], preferred_element_type=jnp.float32)
    z = _silu(z + b1_ref[...])
    col = lax.broadcasted_iota(jnp.int32, (1, 2 * D), 1)
    left = (col < D).astype(jnp.float32)
    sl = jnp.sum(z * left, axis=1, keepdims=True)
    st = jnp.sum(z, axis=1, keepdims=True)
    mean = (sl / D) * left + ((st - sl) / D) * (1.0 - left)
    c = z - mean
    c2 = c * c
    vl = jnp.sum(c2 * left, axis=1, keepdims=True)
    vt = jnp.sum(c2, axis=1, keepdims=True)
    var = (vl / D) * left + ((vt - vl) / D) * (1.0 - left)
    z = c * lax.rsqrt(var + 1e-5)
    z = _silu(jnp.dot(z, w2_ref[...], preferred_element_type=jnp.float32)
              + b2_ref[...])
    out_ref[...] = (jnp.dot(z, w3_ref[...], preferred_element_type=jnp.float32)
                    + b3_ref[...])


def _mlp(e0, e1, w1a, w1b, w2bd, w3bd, b1, b2, b3):
    grid = (B // _BS,)
    row_spec = pl.BlockSpec((_BS, H), lambda i: (i, 0))
    w_spec = pl.BlockSpec((H, H), lambda i: (0, 0))
    b_spec = pl.BlockSpec((1, H), lambda i: (0, 0))
    return pl.pallas_call(
        _mlp_body,
        grid=grid,
        in_specs=[row_spec, row_spec, w_spec, w_spec, w_spec, w_spec,
                  b_spec, b_spec, b_spec],
        out_specs=pl.BlockSpec((_BS, H), lambda i: (i, 0)),
        out_shape=jax.ShapeDtypeStruct((B, H), jnp.float32),
    )(e0, e1, w1a, w1b, w2bd, w3bd, b1, b2, b3)


def kernel(x, E0, E1,
           W1_0, b1_0, W2_0, b2_0, W3_0, b3_0,
           W1_1, b1_1, W2_1, b2_1, W3_1, b3_1):
    x0 = x[:, 0].reshape(_NW * _NCH, _CHUNK)
    x1 = x[:, 1].reshape(_NW * _NCH, _CHUNK)
    e0, e1 = _sc_gather(x0, x1, E0, E1)

    w1 = jnp.concatenate([W1_0.T, W1_1.T], axis=1)
    w1a, w1b = w1[:H], w1[H:]
    zblk = jnp.zeros((D, D), jnp.float32)
    w2bd = jnp.block([[W2_0.T, zblk], [zblk, W2_1.T]])
    w3bd = jnp.block([[W3_0.T, zblk], [zblk, W3_1.T]])
    b1 = jnp.concatenate([b1_0, b1_1]).reshape(1, 2 * D)
    b2 = jnp.concatenate([b2_0, b2_1]).reshape(1, 2 * D)
    b3 = jnp.concatenate([b3_0, b3_1]).reshape(1, 2 * D)

    return _mlp(e0, e1, w1a, w1b, w2bd, w3bd, b1, b2, b3)


# speedup vs baseline: 3.6718x; 3.6718x over previous
import functools

import jax
import jax.numpy as jnp
from jax import lax
from jax.experimental import pallas as pl
from jax.experimental.pallas import tpu as pltpu
from jax.experimental.pallas import tpu_sc as plsc

B = 16384
V = 100000
H = 128
D = 64


_NC = 2
_NS = 16
_NW = _NC * _NS
_BPW = B // _NW
_CHUNK = 128
_NCH = _BPW // _CHUNK


def _gather_body(x0_hbm, x1_hbm, e0_hbm, e1_hbm, out0, out1,
                 idx0_v, idx1_v, r0, r1, r2, r3, sem):
    wid = lax.axis_index("s") * _NC + lax.axis_index("c")
    rows = (r0, r1, r2, r3)
    pltpu.sync_copy(x0_hbm.at[pl.ds(wid * _NCH, _NCH)], idx0_v)
    pltpu.sync_copy(x1_hbm.at[pl.ds(wid * _NCH, _NCH)], idx1_v)
    base = wid * _BPW
    for table, idx_v, out in ((e0_hbm, idx0_v, out0), (e1_hbm, idx1_v, out1)):
        copies = [
            pltpu.async_copy(table.at[idx_v.at[j]], rows[j], sem)
            for j in range(_NCH)
        ]
        for c in copies:
            c.wait()
        for j in range(_NCH):
            pltpu.sync_copy(rows[j], out.at[pl.ds(base + j * _CHUNK, _CHUNK)])


@functools.cache
def _make_sc_gather():
    return pl.kernel(
        _gather_body,
        mesh=plsc.VectorSubcoreMesh(core_axis_name="c", subcore_axis_name="s"),
        out_type=[
            jax.ShapeDtypeStruct((B, H), jnp.float32),
            jax.ShapeDtypeStruct((B, H), jnp.float32),
        ],
        scratch_types=[
            pltpu.VMEM((_NCH, _CHUNK), jnp.int32),
            pltpu.VMEM((_NCH, _CHUNK), jnp.int32),
            pltpu.VMEM((_CHUNK, H), jnp.float32),
            pltpu.VMEM((_CHUNK, H), jnp.float32),
            pltpu.VMEM((_CHUNK, H), jnp.float32),
            pltpu.VMEM((_CHUNK, H), jnp.float32),
            pltpu.SemaphoreType.DMA,
        ],
    )


def _sc_gather(x0, x1, E0, E1):
    return _make_sc_gather()(x0, x1, E0, E1)


_BS = 2048


def _silu(v):
    return v * jax.nn.sigmoid(v)


def _mlp_body(e0_ref, e1_ref, w1a_ref, w1b_ref, w2_ref, w3_ref,
              b1_ref, b2_ref, b3_ref, out_ref):
    z = jnp.dot(e0_ref[...], w1a_ref[...], preferred_element_type=jnp.float32)
    z += jnp.dot(e1_ref[...], w1b_ref[...], preferred_element_type=jnp.float32)
    z = _silu(z + b1_ref[...])
    col = lax.broadcasted_iota(jnp.int32, (1, 2 * D), 1)
    left = (col < D).astype(jnp.float32)
    sl = jnp.sum(z * left, axis=1, keepdims=True)
    st = jnp.sum(z, axis=1, keepdims=True)
    mean = (sl / D) * left + ((st - sl) / D) * (1.0 - left)
    c = z - mean
    c2 = c * c
    vl = jnp.sum(c2 * left, axis=1, keepdims=True)
    vt = jnp.sum(c2, axis=1, keepdims=True)
    var = (vl / D) * left + ((vt - vl) / D) * (1.0 - left)
    z = c * lax.rsqrt(var + 1e-5)
    z = _silu(jnp.dot(z, w2_ref[...], preferred_element_type=jnp.float32)
              + b2_ref[...])
    out_ref[...] = (jnp.dot(z, w3_ref[...], preferred_element_type=jnp.float32)
                    + b3_ref[...])


def _mlp(e0, e1, w1a, w1b, w2bd, w3bd, b1, b2, b3):
    grid = (B // _BS,)
    row_spec = pl.BlockSpec((_BS, H), lambda i: (i, 0))
    w_spec = pl.BlockSpec((H, H), lambda i: (0, 0))
    b_spec = pl.BlockSpec((1, H), lambda i: (0, 0))
    return pl.pallas_call(
        _mlp_body,
        grid=grid,
        in_specs=[row_spec, row_spec, w_spec, w_spec, w_spec, w_spec,
                  b_spec, b_spec, b_spec],
        out_specs=pl.BlockSpec((_BS, H), lambda i: (i, 0)),
        out_shape=jax.ShapeDtypeStruct((B, H), jnp.float32),
    )(e0, e1, w1a, w1b, w2bd, w3bd, b1, b2, b3)


def kernel(x, E0, E1,
           W1_0, b1_0, W2_0, b2_0, W3_0, b3_0,
           W1_1, b1_1, W2_1, b2_1, W3_1, b3_1):
    x0 = x[:, 0].reshape(_NW * _NCH, _CHUNK)
    x1 = x[:, 1].reshape(_NW * _NCH, _CHUNK)
    e0, e1 = _sc_gather(x0, x1, E0, E1)

    w1 = jnp.concatenate([W1_0.T, W1_1.T], axis=1)
    w1a, w1b = w1[:H], w1[H:]
    zblk = jnp.zeros((D, D), jnp.float32)
    w2bd = jnp.block([[W2_0.T, zblk], [zblk, W2_1.T]])
    w3bd = jnp.block([[W3_0.T, zblk], [zblk, W3_1.T]])
    b1 = jnp.concatenate([b1_0, b1_1]).reshape(1, 2 * D)
    b2 = jnp.concatenate([b2_0, b2_1]).reshape(1, 2 * D)
    b3 = jnp.concatenate([b3_0, b3_1]).reshape(1, 2 * D)

    return _mlp(e0, e1, w1a, w1b, w2bd, w3bd, b1, b2, b3)
